# baseline (speedup 1.0000x reference)
import functools

import jax
import jax.numpy as jnp
from jax import lax
from jax.experimental import pallas as pl
from jax.experimental.pallas import tpu as pltpu
from jax.experimental.pallas import tpu_sc as plsc

N_USERS_K = 5000
N_ITEMS_K = 5000
N_TOT = N_USERS_K + N_ITEMS_K
D = 128
E_EDGES = 320000

NS = 16
CHUNK = 128
CPT = 160
NCH = NS * CPT
E_PAD = NCH * CHUNK
ROWS_PER_TILE = 624


def _sc_round_body(emb_hbm, ed_hbm, w_hbm, out_hbm,
                   acc, e0, e1, e2, e3, w0, w1, rows0, rows1,
                   gsem0, gsem1, ssem0, ssem1,
                   esem0, esem1, esem2, esem3, wsem0, wsem1):
    cid = lax.axis_index("c")
    sid = lax.axis_index("s")
    rows = (rows0, rows1)
    eb = (e0, e1, e2, e3)
    gsem = (gsem0, gsem1)
    ssem = (ssem0, ssem1)
    esem = (esem0, esem1, esem2, esem3)
    wb = (w0, w1)
    wsem = (wsem0, wsem1)
    base_row = sid * ROWS_PER_TILE
    tail_base = NS * ROWS_PER_TILE
    tail_rows = N_TOT - tail_base
    cbase = sid * CPT

    def start_e(j, e):
        pltpu.async_copy(ed_hbm.at[cbase + j], eb[e], esem[e])

    def wait_e(e):
        pltpu.make_async_copy(ed_hbm.at[0], eb[e], esem[e]).wait()

    def start_w(j, b):
        pltpu.async_copy(w_hbm.at[cbase + j], wb[b].at[0], wsem[b])

    def wait_w(b):
        pltpu.make_async_copy(w_hbm.at[0], wb[b].at[0], wsem[b]).wait()

    def start_gather(j, b, e):
        pltpu.async_copy(emb_hbm.at[eb[e].at[0]], rows[b], gsem[b])

    def wait_gather(b):
        pltpu.make_async_copy(emb_hbm.at[e0.at[0]], rows[b], gsem[b]).wait()

    def start_scatter(b, e):
        pltpu.async_copy(rows[b], acc.at[eb[e].at[1]], ssem[b], add=True)

    def wait_scatter(b):
        pltpu.make_async_copy(rows[b], acc.at[e0.at[1]], ssem[b]).wait()

    def scale(b):
        rv = rows[b]
        wrow = wb[b]

        def scale_group(g, c2):
            wvec = wrow[0, pl.ds(g * 16, 16)]
            for k in range(16):
                we = wvec[k]
                row = g * 16 + k
                for l in range(D // 16):
                    rv[row, pl.ds(l * 16, 16)] = (
                        rv[row, pl.ds(l * 16, 16)] * we)
            return c2
        lax.fori_loop(0, CHUNK // 16, scale_group, 0)

    @pl.when(cid == 0)
    def _zero_phase():
        def zero_rows(r, carry):
            for l in range(D // 16):
                rows0[r, pl.ds(l * 16, 16)] = jnp.zeros((16,), jnp.float32)
            return carry
        lax.fori_loop(0, CHUNK, zero_rows, 0)
        for k in range(-(-ROWS_PER_TILE // CHUNK)):
            nr = min(CHUNK, ROWS_PER_TILE - k * CHUNK)
            pltpu.sync_copy(rows0.at[pl.ds(0, nr)],
                            acc.at[pl.ds(base_row + k * CHUNK, nr)])

        @pl.when(sid == 0)
        def _zero_tail():
            pltpu.sync_copy(rows0.at[pl.ds(0, tail_rows)],
                            acc.at[pl.ds(tail_base, tail_rows)])
    plsc.subcore_barrier()

    @pl.when(cid == 0)
    def _edge_phase():
        start_e(0, 0)
        start_e(1, 1)
        start_w(0, 0)
        wait_e(0)
        start_gather(0, 0, 0)
        start_e(2, 2)
        wait_gather(0)
        wait_e(1)
        start_gather(1, 1, 1)
        start_e(3, 3)
        start_w(1, 1)
        wait_w(0)
        scale(0)
        start_scatter(0, 0)
        wait_gather(1)
        wait_scatter(0)
        wait_e(2)
        start_gather(2, 0, 2)
        start_w(2, 0)
        wait_w(1)
        scale(1)
        start_scatter(1, 1)

        def step(j, b, e):
            wait_gather(b)
            wait_scatter(1 - b)
            start_e(j + 2, (e + 2) % 4)
            wait_e((e + 1) % 4)
            start_gather(j + 1, 1 - b, (e + 1) % 4)
            start_w(j + 1, 1 - b)
            wait_w(b)
            scale(b)
            start_scatter(b, e)

        def body(t, carry):
            step(4 * t + 2, 0, 2)
            step(4 * t + 3, 1, 3)
            step(4 * t + 4, 0, 0)
            step(4 * t + 5, 1, 1)
            return carry
        lax.fori_loop(0, (CPT - 4) // 4, body, 0)

        wait_gather(0)
        wait_scatter(1)
        wait_e(3)
        start_gather(CPT - 1, 1, 3)
        start_w(CPT - 1, 1)
        wait_w(0)
        scale(0)
        start_scatter(0, 2)
        wait_gather(1)
        wait_scatter(0)
        wait_w(1)
        scale(1)
        start_scatter(1, 3)
        wait_scatter(1)
    plsc.subcore_barrier()

    @pl.when(cid == 0)
    def _readout():
        pltpu.sync_copy(acc.at[pl.ds(base_row, ROWS_PER_TILE)],
                        out_hbm.at[pl.ds(base_row, ROWS_PER_TILE)])

        @pl.when(sid == 0)
        def _write_tail():
            pltpu.sync_copy(acc.at[pl.ds(tail_base, tail_rows)],
                            out_hbm.at[pl.ds(tail_base, tail_rows)])


@jax.jit
def _sc_round(emb, edata, w2d):
    mesh = plsc.VectorSubcoreMesh(core_axis_name="c", subcore_axis_name="s")
    return pl.kernel(
        _sc_round_body,
        out_type=jax.ShapeDtypeStruct((N_TOT, D), jnp.float32),
        mesh=mesh,
        scratch_types=[
            pltpu.VMEM_SHARED((N_TOT, D), jnp.float32),
            pltpu.VMEM((2, CHUNK), jnp.int32),
            pltpu.VMEM((2, CHUNK), jnp.int32),
            pltpu.VMEM((2, CHUNK), jnp.int32),
            pltpu.VMEM((2, CHUNK), jnp.int32),
            pltpu.VMEM((1, CHUNK), jnp.float32),
            pltpu.VMEM((1, CHUNK), jnp.float32),
            pltpu.VMEM((CHUNK, D), jnp.float32),
            pltpu.VMEM((CHUNK, D), jnp.float32),
            pltpu.SemaphoreType.DMA,
            pltpu.SemaphoreType.DMA,
            pltpu.SemaphoreType.DMA,
            pltpu.SemaphoreType.DMA,
            pltpu.SemaphoreType.DMA,
            pltpu.SemaphoreType.DMA,
            pltpu.SemaphoreType.DMA,
            pltpu.SemaphoreType.DMA,
            pltpu.SemaphoreType.DMA,
            pltpu.SemaphoreType.DMA,
        ],
    )(emb, edata, w2d)


def _final_body(e0_ref, e1_ref, e2_ref, o_ref):
    o_ref[...] = (e0_ref[...] + e1_ref[...] + e2_ref[...]) * (1.0 / 3.0)


_TC_BLK = 1000


@jax.jit
def _final(emb0, emb1, emb2):
    spec = pl.BlockSpec((_TC_BLK, D), lambda i: (i, 0))
    return pl.pallas_call(
        _final_body,
        grid=(N_TOT // _TC_BLK,),
        in_specs=[spec] * 3,
        out_specs=spec,
        out_shape=jax.ShapeDtypeStruct((N_TOT, D), jnp.float32),
    )(emb0, emb1, emb2)


def kernel(edge_index, edge_weight, user_emb, item_emb):
    emb0 = jnp.concatenate([user_emb, item_emb], axis=0)
    dst = edge_index[0]
    src = edge_index[1]
    pad = E_PAD - E_EDGES
    src2d = jnp.pad(src, (0, pad)).reshape(NCH, CHUNK)
    dst2d = jnp.pad(dst, (0, pad)).reshape(NCH, CHUNK)
    w2d = jnp.pad(edge_weight, (0, pad)).reshape(NCH, CHUNK)
    edata = jnp.stack([src2d, dst2d], axis=1)

    emb1 = _sc_round(emb0, edata, w2d)
    emb2 = _sc_round(emb1, edata, w2d)
    out = _final(emb0, emb1, emb2)
    return (out[:N_USERS_K], out[N_USERS_K:])


# speedup vs baseline: 1.1741x; 1.1741x over previous
import functools

import jax
import jax.numpy as jnp
from jax import lax
from jax.experimental import pallas as pl
from jax.experimental.pallas import tpu as pltpu
from jax.experimental.pallas import tpu_sc as plsc

N_USERS_K = 5000
N_ITEMS_K = 5000
N_TOT = N_USERS_K + N_ITEMS_K
D = 128
E_EDGES = 320000

NS = 16
CHUNK = 128
CPT = 160
HPT = CPT // 2
NCH = NS * CPT
E_PAD = NCH * CHUNK
ROWS_PER_TILE = 624


def _sc_round_body(emb_hbm, spk_hbm, dst_hbm, w_hbm, out_hbm,
                   acc, slab, i0, i1, d0, d1, w0, w1, rows0, rows1,
                   gsem0, gsem1, ssem0, ssem1,
                   dsem0, dsem1, wsem0, wsem1):
    cid = lax.axis_index("c")
    sid = lax.axis_index("s")
    rows = (rows0, rows1)
    ib = (i0, i1)
    db = (d0, d1)
    wb = (w0, w1)
    gsem = (gsem0, gsem1)
    ssem = (ssem0, ssem1)
    dsem = (dsem0, dsem1)
    wsem = (wsem0, wsem1)
    base_row = sid * ROWS_PER_TILE
    tail_base = NS * ROWS_PER_TILE
    tail_rows = N_TOT - tail_base
    cbase = sid * CPT

    def start_dst(j, b):
        pltpu.async_copy(dst_hbm.at[cbase + j], db[b].at[0], dsem[b])

    def wait_dst(b):
        pltpu.make_async_copy(dst_hbm.at[0], db[b].at[0], dsem[b]).wait()

    def start_w(j, b):
        pltpu.async_copy(w_hbm.at[cbase + j], wb[b].at[0], wsem[b])

    def wait_w(b):
        pltpu.make_async_copy(w_hbm.at[0], wb[b].at[0], wsem[b]).wait()

    def build_idx(row, half, b):
        for g in range(4):
            v = slab[row, pl.ds(64 * half + g * 16, 16)]
            ib[b][0, pl.ds(g * 16, 16)] = v & 0xFFFF
            ib[b][0, pl.ds(64 + g * 16, 16)] = v >> 16

    def start_gather(b):
        pltpu.async_copy(emb_hbm.at[ib[b].at[0]], rows[b], gsem[b])

    def wait_gather(b):
        pltpu.make_async_copy(emb_hbm.at[i0.at[0]], rows[b], gsem[b]).wait()

    def start_scatter(b):
        pltpu.async_copy(rows[b], acc.at[db[b].at[0]], ssem[b], add=True)

    def wait_scatter(b):
        pltpu.make_async_copy(rows[b], acc.at[d0.at[0]], ssem[b]).wait()

    def scale(b):
        rv = rows[b]
        wrow = wb[b]

        def scale_group(g, c2):
            wvec = wrow[0, pl.ds(g * 16, 16)]
            for k in range(16):
                we = wvec[k]
                row = g * 16 + k
                for l in range(D // 16):
                    rv[row, pl.ds(l * 16, 16)] = (
                        rv[row, pl.ds(l * 16, 16)] * we)
            return c2
        lax.fori_loop(0, CHUNK // 16, scale_group, 0)

    @pl.when(cid == 0)
    def _zero_phase():
        def zero_rows(r, carry):
            for l in range(D // 16):
                rows0[r, pl.ds(l * 16, 16)] = jnp.zeros((16,), jnp.float32)
            return carry
        lax.fori_loop(0, CHUNK, zero_rows, 0)
        for k in range(-(-ROWS_PER_TILE // CHUNK)):
            nr = min(CHUNK, ROWS_PER_TILE - k * CHUNK)
            pltpu.sync_copy(rows0.at[pl.ds(0, nr)],
                            acc.at[pl.ds(base_row + k * CHUNK, nr)])

        @pl.when(sid == 0)
        def _zero_tail():
            pltpu.sync_copy(rows0.at[pl.ds(0, tail_rows)],
                            acc.at[pl.ds(tail_base, tail_rows)])
    plsc.subcore_barrier()

    @pl.when(cid == 0)
    def _edge_phase():
        pltpu.sync_copy(spk_hbm.at[pl.ds(sid * HPT, HPT)], slab)

        start_dst(0, 0)
        start_w(0, 0)
        build_idx(0, 0, 0)
        start_gather(0)
        build_idx(0, 1, 1)
        start_gather(1)
        start_dst(1, 1)
        start_w(1, 1)
        wait_gather(0)
        wait_w(0)
        scale(0)
        wait_dst(0)
        start_scatter(0)

        def step(j, row1, b):
            wait_gather(b)
            wait_scatter(1 - b)
            build_idx(row1, 1 - b, 1 - b)
            start_gather(1 - b)
            start_dst(j + 1, 1 - b)
            start_w(j + 1, 1 - b)
            wait_w(b)
            scale(b)
            wait_dst(b)
            start_scatter(b)

        def body(t, carry):
            step(2 * t + 1, t + 1, 1)
            step(2 * t + 2, t + 1, 0)
            return carry
        lax.fori_loop(0, (CPT - 2) // 2, body, 0)

        wait_gather(1)
        wait_scatter(0)
        wait_w(1)
        scale(1)
        wait_dst(1)
        start_scatter(1)
        wait_scatter(1)
    plsc.subcore_barrier()

    @pl.when(cid == 0)
    def _readout():
        pltpu.sync_copy(acc.at[pl.ds(base_row, ROWS_PER_TILE)],
                        out_hbm.at[pl.ds(base_row, ROWS_PER_TILE)])

        @pl.when(sid == 0)
        def _write_tail():
            pltpu.sync_copy(acc.at[pl.ds(tail_base, tail_rows)],
                            out_hbm.at[pl.ds(tail_base, tail_rows)])


@jax.jit
def _sc_round(emb, spk, dst2d, w2d):
    mesh = plsc.VectorSubcoreMesh(core_axis_name="c", subcore_axis_name="s")
    return pl.kernel(
        _sc_round_body,
        out_type=jax.ShapeDtypeStruct((N_TOT, D), jnp.float32),
        mesh=mesh,
        scratch_types=[
            pltpu.VMEM_SHARED((N_TOT, D), jnp.float32),
            pltpu.VMEM((HPT, CHUNK), jnp.int32),
            pltpu.VMEM((1, CHUNK), jnp.int32),
            pltpu.VMEM((1, CHUNK), jnp.int32),
            pltpu.VMEM((1, CHUNK), jnp.int32),
            pltpu.VMEM((1, CHUNK), jnp.int32),
            pltpu.VMEM((1, CHUNK), jnp.float32),
            pltpu.VMEM((1, CHUNK), jnp.float32),
            pltpu.VMEM((CHUNK, D), jnp.float32),
            pltpu.VMEM((CHUNK, D), jnp.float32),
            pltpu.SemaphoreType.DMA,
            pltpu.SemaphoreType.DMA,
            pltpu.SemaphoreType.DMA,
            pltpu.SemaphoreType.DMA,
            pltpu.SemaphoreType.DMA,
            pltpu.SemaphoreType.DMA,
            pltpu.SemaphoreType.DMA,
            pltpu.SemaphoreType.DMA,
        ],
    )(emb, spk, dst2d, w2d)


def _final_body(e0_ref, e1_ref, e2_ref, o_ref):
    o_ref[...] = (e0_ref[...] + e1_ref[...] + e2_ref[...]) * (1.0 / 3.0)


_TC_BLK = 1000


@jax.jit
def _final(emb0, emb1, emb2):
    spec = pl.BlockSpec((_TC_BLK, D), lambda i: (i, 0))
    return pl.pallas_call(
        _final_body,
        grid=(N_TOT // _TC_BLK,),
        in_specs=[spec] * 3,
        out_specs=spec,
        out_shape=jax.ShapeDtypeStruct((N_TOT, D), jnp.float32),
    )(emb0, emb1, emb2)


def kernel(edge_index, edge_weight, user_emb, item_emb):
    emb0 = jnp.concatenate([user_emb, item_emb], axis=0)
    dst = edge_index[0]
    src = edge_index[1]
    pad = E_PAD - E_EDGES
    src2d = jnp.pad(src, (0, pad)).reshape(NCH, CHUNK)
    dst2d = jnp.pad(dst, (0, pad)).reshape(NCH, CHUNK)
    w2d = jnp.pad(edge_weight, (0, pad)).reshape(NCH, CHUNK)
    spk = (src2d[:, :64] | (src2d[:, 64:] << 16)).reshape(NCH // 2, CHUNK)

    emb1 = _sc_round(emb0, spk, dst2d, w2d)
    emb2 = _sc_round(emb1, spk, dst2d, w2d)
    out = _final(emb0, emb1, emb2)
    return (out[:N_USERS_K], out[N_USERS_K:])
